# baseline (speedup 1.0000x reference)
import functools

import jax
import jax.numpy as jnp
from jax import lax
from jax.experimental import pallas as pl
from jax.experimental.pallas import tpu as pltpu
from jax.experimental.pallas import tpu_sc as plsc

B, T, D = 16, 2048, 384
MAXLEN = 4096
CHUNK = 32
NCH_ROW = MAXLEN // CHUNK
NCH = NCH_ROW // 2
NBUF = 8
GA = 7
DV = D // 16


def _lr_body(x_hbm, dur_hbm, out_hbm, mel_hbm,
             dur_v, gidx_v, bufs, zbuf, mel_v, sem_in, sem_g, sem_w, sem_z):
    b = lax.axis_index("s")
    half = lax.axis_index("c")
    out_base = b * MAXLEN
    base_row = b * T

    dur_cp = pltpu.async_copy(dur_hbm.at[b], dur_v, sem_in)

    def _zb(r, _):
        for q in range(DV):
            zbuf[r, pl.ds(q * 16, 16)] = jnp.zeros((16,), jnp.float32)
        return 0
    lax.fori_loop(0, CHUNK, _zb, 0)
    dur_cp.wait()

    lane = lax.iota(jnp.int32, 16)

    def _fr(c):
        return pl.multiple_of((2 * c + half) * CHUNK, CHUNK)

    def _gather_start(c):
        pltpu.async_copy(
            x_hbm.at[gidx_v.at[pl.ds(_fr(c), CHUNK)]], bufs.at[c % NBUF],
            sem_g.at[c % NBUF])

    def _gather_wait(c):
        pltpu.make_async_copy(
            x_hbm.at[gidx_v.at[pl.ds(_fr(c), CHUNK)]], bufs.at[c % NBUF],
            sem_g.at[c % NBUF]).wait()

    def _write_start(c):
        pltpu.async_copy(
            bufs.at[c % NBUF], out_hbm.at[pl.ds(out_base + _fr(c), CHUNK)],
            sem_w.at[c % NBUF])

    def _write_wait(c):
        pltpu.make_async_copy(
            bufs.at[c % NBUF], out_hbm.at[pl.ds(out_base + _fr(c), CHUNK)],
            sem_w.at[c % NBUF]).wait()

    def _prime(k, _):
        _gather_start(k)
        return 0

    def _build(i, st):
        carry, primed = st
        v = dur_v[pl.ds(i * 16, 16)]
        v = jnp.maximum(v, 0)
        c = plsc.cumsum(v) + carry
        s = c - v
        val = base_row + i * 16 + lane
        for k in range(3):
            p = s + k
            m = (v > k) & (p < MAXLEN)
            plsc.store_scatter(gidx_v, (p,), val, mask=m)
        new_carry = c[15]
        safe = jnp.minimum(new_carry, MAXLEN) // CHUNK
        sn = jnp.minimum(GA, jnp.maximum((safe - half + 1) // 2, 0))

        @pl.when(primed < sn)
        def _():
            _gather_start(primed)
        return (new_carry, jnp.where(primed < sn, primed + 1, primed))

    mel, p0 = lax.fori_loop(0, T // 16, _build, (jnp.int32(0), jnp.int32(0)))

    bound = jnp.minimum(mel, MAXLEN)
    nfull_row = bound // CHUNK
    rem_row = bound % CHUNK
    ng_row = nfull_row + jnp.where(rem_row > 0, 1, 0)
    n_g = jnp.maximum((ng_row - half + 1) // 2, 0)

    ceil_f = ng_row * CHUNK
    for k in range(CHUNK // 16):
        p = bound + k * 16 + lane
        plsc.store_scatter(gidx_v, (p,), jnp.full((16,), base_row, jnp.int32),
                           mask=p < ceil_f)

    @pl.when(half == 0)
    def _():
        mel_v[...] = jnp.full((16,), mel, jnp.int32)
        pltpu.async_copy(mel_v, mel_hbm.at[b], sem_in)

    @pl.when(n_g > 0)
    def _():
        lax.fori_loop(p0, jnp.minimum(GA, n_g), _prime, 0)

        def _step(c, _):
            @pl.when(c + GA < n_g)
            def _():
                @pl.when(c >= 1)
                def _():
                    _write_wait(c - 1)
                _gather_start(c + GA)

            _gather_wait(c)

            zero_from = jnp.where(2 * c + half == nfull_row, rem_row, CHUNK)
            p = c % NBUF

            def _zrow(r, _):
                for q in range(DV):
                    bufs[p, r, pl.ds(q * 16, 16)] = jnp.zeros(
                        (16,), jnp.float32)
                return 0
            lax.fori_loop(zero_from, CHUNK, _zrow, 0)

            _write_start(c)
            return 0

        lax.fori_loop(0, n_g, _step, 0)

    def _zfire(c, _):
        pltpu.async_copy(
            zbuf, out_hbm.at[pl.ds(out_base + _fr(c), CHUNK)], sem_z)
        return 0
    lax.fori_loop(n_g, NCH, _zfire, 0)

    @pl.when(n_g > 0)
    def _():
        def _drain(k, _):
            _write_wait(k)
            return 0
        lax.fori_loop(jnp.maximum(n_g - NBUF, 0), n_g, _drain, 0)

    def _zdrain(c, _):
        pltpu.make_async_copy(
            zbuf, out_hbm.at[pl.ds(out_base + _fr(c), CHUNK)],
            sem_z).wait()
        return 0
    lax.fori_loop(n_g, NCH, _zdrain, 0)

    @pl.when(half == 0)
    def _():
        pltpu.make_async_copy(mel_v, mel_hbm.at[b], sem_in).wait()


@jax.jit
def _lr_call(x_flat, dur):
    mesh = plsc.VectorSubcoreMesh(
        core_axis_name="c", subcore_axis_name="s",
        num_cores=2, num_subcores=16)
    f = pl.kernel(
        _lr_body,
        out_type=(
            jax.ShapeDtypeStruct((B * MAXLEN, D), jnp.float32),
            jax.ShapeDtypeStruct((B, 16), jnp.int32),
        ),
        mesh=mesh,
        compiler_params=pltpu.CompilerParams(needs_layout_passes=False),
        scratch_types=[
            pltpu.VMEM((T,), jnp.int32),
            pltpu.VMEM((MAXLEN,), jnp.int32),
            pltpu.VMEM((NBUF, CHUNK, D), jnp.float32),
            pltpu.VMEM((CHUNK, D), jnp.float32),
            pltpu.VMEM((16,), jnp.int32),
            pltpu.SemaphoreType.DMA,
            pltpu.SemaphoreType.DMA((NBUF,)),
            pltpu.SemaphoreType.DMA((NBUF,)),
            pltpu.SemaphoreType.DMA,
        ],
    )
    return f(x_flat, dur)


def kernel(x, duration, max_len):
    del max_len
    x_flat = x.reshape(B * T, D)
    dur = duration.astype(jnp.int32)
    out_flat, mel_pad = _lr_call(x_flat, dur)
    out = out_flat.reshape(B, MAXLEN, D)
    mel_len = mel_pad[:, 0].astype(jnp.int64)
    return out, mel_len


# speedup vs baseline: 1.0317x; 1.0317x over previous
import functools

import jax
import jax.numpy as jnp
from jax import lax
from jax.experimental import pallas as pl
from jax.experimental.pallas import tpu as pltpu
from jax.experimental.pallas import tpu_sc as plsc

B, T, D = 16, 2048, 384
MAXLEN = 4096
CHUNK = 32
NCH_ROW = MAXLEN // CHUNK
NCH = NCH_ROW // 2
NBUF = 8
GA = 7
DV = D // 16


def _lr_body(x_hbm, dur_hbm, out_hbm, mel_hbm,
             dur_v, gidx_v, bufs, zbuf, mel_v, sem_in, sem_g, sem_w, sem_z):
    b = lax.axis_index("s")
    half = lax.axis_index("c")
    out_base = b * MAXLEN
    base_row = b * T

    dur_cp = pltpu.async_copy(dur_hbm.at[b], dur_v, sem_in)

    def _zb(r, _):
        for q in range(DV):
            zbuf[r, pl.ds(q * 16, 16)] = jnp.zeros((16,), jnp.float32)
        return 0
    lax.fori_loop(0, CHUNK, _zb, 0)
    dur_cp.wait()

    lane = lax.iota(jnp.int32, 16)

    def _build(i, carry):
        v = dur_v[pl.ds(i * 16, 16)]
        v = jnp.maximum(v, 0)
        c = plsc.cumsum(v) + carry
        s = c - v
        val = base_row + i * 16 + lane
        for k in range(3):
            p = s + k
            m = (v > k) & (p < MAXLEN)
            plsc.store_scatter(gidx_v, (p,), val, mask=m)
        return c[15]

    def _fr(c):
        return pl.multiple_of((2 * c + half) * CHUNK, CHUNK)

    def _gather_start(c):
        pltpu.async_copy(
            x_hbm.at[gidx_v.at[pl.ds(_fr(c), CHUNK)]], bufs.at[c % NBUF],
            sem_g.at[c % NBUF])

    def _gather_wait(c):
        pltpu.make_async_copy(
            x_hbm.at[gidx_v.at[pl.ds(_fr(c), CHUNK)]], bufs.at[c % NBUF],
            sem_g.at[c % NBUF]).wait()

    def _write_start(c):
        pltpu.async_copy(
            bufs.at[c % NBUF], out_hbm.at[pl.ds(out_base + _fr(c), CHUNK)],
            sem_w.at[c % NBUF])

    def _write_wait(c):
        pltpu.make_async_copy(
            bufs.at[c % NBUF], out_hbm.at[pl.ds(out_base + _fr(c), CHUNK)],
            sem_w.at[c % NBUF]).wait()

    def _prime(k, _):
        _gather_start(k)
        return 0

    mel1 = lax.fori_loop(0, T // 32, _build, jnp.int32(0))
    safe_ng_row = jnp.minimum(mel1, MAXLEN) // CHUNK
    p0 = jnp.minimum(GA, jnp.maximum((safe_ng_row - half + 1) // 2, 0))
    lax.fori_loop(0, p0, _prime, 0)
    mel = lax.fori_loop(T // 32, T // 16, _build, mel1)

    bound = jnp.minimum(mel, MAXLEN)
    nfull_row = bound // CHUNK
    rem_row = bound % CHUNK
    ng_row = nfull_row + jnp.where(rem_row > 0, 1, 0)
    n_g = jnp.maximum((ng_row - half + 1) // 2, 0)

    ceil_f = ng_row * CHUNK
    for k in range(CHUNK // 16):
        p = bound + k * 16 + lane
        plsc.store_scatter(gidx_v, (p,), jnp.full((16,), base_row, jnp.int32),
                           mask=p < ceil_f)

    @pl.when(half == 0)
    def _():
        mel_v[...] = jnp.full((16,), mel, jnp.int32)
        pltpu.async_copy(mel_v, mel_hbm.at[b], sem_in)

    @pl.when(n_g > 0)
    def _():
        lax.fori_loop(p0, jnp.minimum(GA, n_g), _prime, 0)

        def _step(c, _):
            @pl.when(c + GA < n_g)
            def _():
                @pl.when(c >= 1)
                def _():
                    _write_wait(c - 1)
                _gather_start(c + GA)

            _gather_wait(c)

            zero_from = jnp.where(2 * c + half == nfull_row, rem_row, CHUNK)
            p = c % NBUF

            def _zrow(r, _):
                for q in range(DV):
                    bufs[p, r, pl.ds(q * 16, 16)] = jnp.zeros(
                        (16,), jnp.float32)
                return 0
            lax.fori_loop(zero_from, CHUNK, _zrow, 0)

            _write_start(c)
            return 0

        lax.fori_loop(0, n_g, _step, 0)

        def _drain(k, _):
            _write_wait(k)
            return 0
        lax.fori_loop(jnp.maximum(n_g - NBUF, 0), n_g, _drain, 0)

    @pl.when(n_g < NCH)
    def _():
        def _zfire(c, _):
            pltpu.async_copy(
                zbuf, out_hbm.at[pl.ds(out_base + _fr(c), CHUNK)], sem_z)
            return 0
        lax.fori_loop(n_g, NCH, _zfire, 0)

        def _zdrain(c, _):
            pltpu.make_async_copy(
                zbuf, out_hbm.at[pl.ds(out_base + _fr(c), CHUNK)],
                sem_z).wait()
            return 0
        lax.fori_loop(n_g, NCH, _zdrain, 0)

    @pl.when(half == 0)
    def _():
        pltpu.make_async_copy(mel_v, mel_hbm.at[b], sem_in).wait()


@jax.jit
def _lr_call(x_flat, dur):
    mesh = plsc.VectorSubcoreMesh(
        core_axis_name="c", subcore_axis_name="s",
        num_cores=2, num_subcores=16)
    f = pl.kernel(
        _lr_body,
        out_type=(
            jax.ShapeDtypeStruct((B * MAXLEN, D), jnp.float32),
            jax.ShapeDtypeStruct((B, 16), jnp.int32),
        ),
        mesh=mesh,
        compiler_params=pltpu.CompilerParams(needs_layout_passes=False),
        scratch_types=[
            pltpu.VMEM((T,), jnp.int32),
            pltpu.VMEM((MAXLEN,), jnp.int32),
            pltpu.VMEM((NBUF, CHUNK, D), jnp.float32),
            pltpu.VMEM((CHUNK, D), jnp.float32),
            pltpu.VMEM((16,), jnp.int32),
            pltpu.SemaphoreType.DMA,
            pltpu.SemaphoreType.DMA((NBUF,)),
            pltpu.SemaphoreType.DMA((NBUF,)),
            pltpu.SemaphoreType.DMA,
        ],
    )
    return f(x_flat, dur)


def kernel(x, duration, max_len):
    del max_len
    x_flat = x.reshape(B * T, D)
    dur = duration.astype(jnp.int32)
    out_flat, mel_pad = _lr_call(x_flat, dur)
    out = out_flat.reshape(B, MAXLEN, D)
    mel_len = mel_pad[:, 0].astype(jnp.int64)
    return out, mel_len


# speedup vs baseline: 1.0352x; 1.0035x over previous
import functools

import jax
import jax.numpy as jnp
from jax import lax
from jax.experimental import pallas as pl
from jax.experimental.pallas import tpu as pltpu
from jax.experimental.pallas import tpu_sc as plsc

B, T, D = 16, 2048, 384
MAXLEN = 4096
CHUNK = 32
NCH_ROW = MAXLEN // CHUNK
NCH = NCH_ROW // 2
NBUF = 8
GA = 7
DV = D // 16


def _lr_body(x_hbm, dur_hbm, out_hbm, mel_hbm,
             dur_v, gidx_v, bufs, zbuf, mel_v, sem_in, sem_g, sem_w, sem_z):
    b = lax.axis_index("s")
    half = lax.axis_index("c")
    out_base = b * MAXLEN
    base_row = b * T

    dur_cp = pltpu.async_copy(dur_hbm.at[b], dur_v, sem_in)

    def _zb(r, _):
        for q in range(DV):
            zbuf[r, pl.ds(q * 16, 16)] = jnp.zeros((16,), jnp.float32)
        return 0
    lax.fori_loop(0, CHUNK, _zb, 0)
    dur_cp.wait()

    lane = lax.iota(jnp.int32, 16)

    def _build(i, carry):
        v = dur_v[pl.ds(i * 16, 16)]
        v = jnp.maximum(v, 0)
        c = plsc.cumsum(v) + carry
        s = c - v
        val = base_row + i * 16 + lane
        for k in range(3):
            p = s + k
            m = (v > k) & (p < MAXLEN)
            plsc.store_scatter(gidx_v, (p,), val, mask=m)
        return c[15]

    def _fr(c):
        return pl.multiple_of((2 * c + half) * CHUNK, CHUNK)

    def _gather_start(c):
        pltpu.async_copy(
            x_hbm.at[gidx_v.at[pl.ds(_fr(c), CHUNK)]], bufs.at[c % NBUF],
            sem_g.at[c % NBUF])

    def _gather_wait(c):
        pltpu.make_async_copy(
            x_hbm.at[gidx_v.at[pl.ds(_fr(c), CHUNK)]], bufs.at[c % NBUF],
            sem_g.at[c % NBUF]).wait()

    def _write_start(c):
        pltpu.async_copy(
            bufs.at[c % NBUF], out_hbm.at[pl.ds(out_base + _fr(c), CHUNK)],
            sem_w.at[c % NBUF])

    def _write_wait(c):
        pltpu.make_async_copy(
            bufs.at[c % NBUF], out_hbm.at[pl.ds(out_base + _fr(c), CHUNK)],
            sem_w.at[c % NBUF]).wait()

    def _prime(k, _):
        _gather_start(k)
        return 0

    mel1 = lax.fori_loop(0, T // 32, _build, jnp.int32(0))
    safe_ng_row = jnp.minimum(mel1, MAXLEN) // CHUNK
    p0 = jnp.minimum(GA, jnp.maximum((safe_ng_row - half + 1) // 2, 0))
    lax.fori_loop(0, p0, _prime, 0)
    mel = lax.fori_loop(T // 32, T // 16, _build, mel1)

    bound = jnp.minimum(mel, MAXLEN)
    nfull_row = bound // CHUNK
    rem_row = bound % CHUNK
    ng_row = nfull_row + jnp.where(rem_row > 0, 1, 0)
    n_g = jnp.maximum((ng_row - half + 1) // 2, 0)

    ceil_f = ng_row * CHUNK
    for k in range(CHUNK // 16):
        p = bound + k * 16 + lane
        plsc.store_scatter(gidx_v, (p,), jnp.full((16,), base_row, jnp.int32),
                           mask=p < ceil_f)

    @pl.when(half == 0)
    def _():
        mel_v[...] = jnp.full((16,), mel, jnp.int32)
        pltpu.async_copy(mel_v, mel_hbm.at[b], sem_in)

    @pl.when(n_g > 0)
    def _():
        lax.fori_loop(p0, jnp.minimum(GA, n_g), _prime, 0)

        def _step(c, _):
            @pl.when(c + GA < n_g)
            def _():
                @pl.when(c >= 1)
                def _():
                    _write_wait(c - 1)
                _gather_start(c + GA)

            _gather_wait(c)

            zero_from = jnp.where(2 * c + half == nfull_row, rem_row, CHUNK)
            p = c % NBUF

            def _zrow(r, _):
                for q in range(DV):
                    bufs[p, r, pl.ds(q * 16, 16)] = jnp.zeros(
                        (16,), jnp.float32)
                return 0
            lax.fori_loop(zero_from, CHUNK, _zrow, 0)

            _write_start(c)
            return 0

        lax.fori_loop(0, n_g, _step, 0)

    def _zfire(c, _):
        pltpu.async_copy(
            zbuf, out_hbm.at[pl.ds(out_base + _fr(c), CHUNK)], sem_z)
        return 0
    lax.fori_loop(n_g, NCH, _zfire, 0)

    @pl.when(n_g > 0)
    def _():
        def _drain(k, _):
            _write_wait(k)
            return 0
        lax.fori_loop(jnp.maximum(n_g - NBUF, 0), n_g, _drain, 0)

    def _zdrain(c, _):
        pltpu.make_async_copy(
            zbuf, out_hbm.at[pl.ds(out_base + _fr(c), CHUNK)],
            sem_z).wait()
        return 0
    lax.fori_loop(n_g, NCH, _zdrain, 0)

    @pl.when(half == 0)
    def _():
        pltpu.make_async_copy(mel_v, mel_hbm.at[b], sem_in).wait()


@jax.jit
def _lr_call(x_flat, dur):
    mesh = plsc.VectorSubcoreMesh(
        core_axis_name="c", subcore_axis_name="s",
        num_cores=2, num_subcores=16)
    f = pl.kernel(
        _lr_body,
        out_type=(
            jax.ShapeDtypeStruct((B * MAXLEN, D), jnp.float32),
            jax.ShapeDtypeStruct((B, 16), jnp.int32),
        ),
        mesh=mesh,
        compiler_params=pltpu.CompilerParams(needs_layout_passes=False),
        scratch_types=[
            pltpu.VMEM((T,), jnp.int32),
            pltpu.VMEM((MAXLEN,), jnp.int32),
            pltpu.VMEM((NBUF, CHUNK, D), jnp.float32),
            pltpu.VMEM((CHUNK, D), jnp.float32),
            pltpu.VMEM((16,), jnp.int32),
            pltpu.SemaphoreType.DMA,
            pltpu.SemaphoreType.DMA((NBUF,)),
            pltpu.SemaphoreType.DMA((NBUF,)),
            pltpu.SemaphoreType.DMA,
        ],
    )
    return f(x_flat, dur)


def kernel(x, duration, max_len):
    del max_len
    x_flat = x.reshape(B * T, D)
    dur = duration.astype(jnp.int32)
    out_flat, mel_pad = _lr_call(x_flat, dur)
    out = out_flat.reshape(B, MAXLEN, D)
    mel_len = mel_pad[:, 0].astype(jnp.int64)
    return out, mel_len
